# baseline (speedup 1.0000x reference)
import functools

import jax
import jax.numpy as jnp
from jax import lax
from jax.experimental import pallas as pl
from jax.experimental.pallas import tpu as pltpu
from jax.experimental.pallas import tpu_sc as plsc

NUM_FIELDS = 26
VOCAB = 100000
EMB_DIM = 32
BATCH = 16384

NC = 2
NS = 16
NW = NC * NS
BPW = BATCH // NW


def _sc_body(idx_hbm, table_hbm, out_hbm, idx_v, buf_v, acc_v, sem):
    c = lax.axis_index("c")
    s = lax.axis_index("s")
    wid = s * NC + c
    base = wid * BPW

    pltpu.sync_copy(idx_hbm.at[wid], idx_v)

    def start_gather(f, slot):
        pltpu.async_copy(table_hbm.at[idx_v.at[f]], buf_v.at[slot], sem)

    def drain(slot):
        pltpu.make_async_copy(
            table_hbm.at[idx_v.at[0]], buf_v.at[slot], sem
        ).wait()

    start_gather(0, 0)

    start_gather(1, 1)
    drain(0)

    def init_body(j, carry):
        acc_v[j, pl.ds(0, 16)] = buf_v[0, j, pl.ds(0, 16)]
        acc_v[j, pl.ds(16, 16)] = buf_v[0, j, pl.ds(16, 16)]
        return carry

    lax.fori_loop(0, BPW, init_body, 0)

    def field_body(f, carry):
        slot = lax.rem(f, 2)
        nslot = lax.rem(f + 1, 2)

        @pl.when(f + 1 < NUM_FIELDS)
        def _():
            start_gather(f + 1, nslot)

        drain(slot)

        def acc_body(j, carry2):
            plsc.addupdate(acc_v.at[j].at[pl.ds(0, 16)],
                           buf_v[slot, j, pl.ds(0, 16)])
            plsc.addupdate(acc_v.at[j].at[pl.ds(16, 16)],
                           buf_v[slot, j, pl.ds(16, 16)])
            return carry2

        lax.fori_loop(0, BPW, acc_body, 0)
        return carry

    lax.fori_loop(1, NUM_FIELDS, field_body, 0)

    pltpu.sync_copy(acc_v, out_hbm.at[pl.ds(base, BPW)])


_emb_call = functools.partial(
    pl.kernel,
    mesh=plsc.VectorSubcoreMesh(
        core_axis_name="c", subcore_axis_name="s", num_cores=NC, num_subcores=NS
    ),
    out_type=jax.ShapeDtypeStruct((BATCH, EMB_DIM), jnp.float32),
    scratch_types=[
        pltpu.VMEM((NUM_FIELDS, BPW), jnp.int32),
        pltpu.VMEM((2, BPW, EMB_DIM), jnp.float32),
        pltpu.VMEM((BPW, EMB_DIM), jnp.float32),
        pltpu.SemaphoreType.DMA,
    ],
    compiler_params=pltpu.CompilerParams(use_tc_tiling_on_sc=False),
)(_sc_body)


@jax.jit
def kernel(g, x, tables):
    x = x.astype(jnp.int32)
    offs = (jnp.arange(NUM_FIELDS, dtype=jnp.int32) * VOCAB)[None, :]
    flat = x + offs
    idx = flat.reshape(NW, BPW, NUM_FIELDS).transpose(0, 2, 1)
    table = tables.reshape(NUM_FIELDS * VOCAB, EMB_DIM)
    return _emb_call(idx, table)


# speedup vs baseline: 1.0784x; 1.0784x over previous
import functools

import jax
import jax.numpy as jnp
from jax import lax
from jax.experimental import pallas as pl
from jax.experimental.pallas import tpu as pltpu
from jax.experimental.pallas import tpu_sc as plsc

NUM_FIELDS = 26
VOCAB = 100000
EMB_DIM = 32
BATCH = 16384

NC = 2
NS = 16
NW = NC * NS
BPW = BATCH // NW
CB = 64
NCHUNK = BPW // CB
ROWS_PER_CHUNK = CB * NUM_FIELDS
VG = ROWS_PER_CHUNK // 16
NSEM = 8
ROW_BYTES = EMB_DIM * 4
SEM_BYTES = (VG // NSEM) * 16 * ROW_BYTES


def _sc_body(idx_hbm, table_hbm, out_hbm, idx_v, buf_v, outb_v, sems):
    c = lax.axis_index("c")
    s = lax.axis_index("s")
    wid = s * NC + c
    base = wid * BPW

    pltpu.sync_copy(idx_hbm.at[wid], idx_v)

    def start_gathers(ch, slot):
        def one(g, carry):
            iv = idx_v[ch, pl.ds(g * 16, 16)]
            pltpu.async_copy(
                table_hbm.at[iv],
                buf_v.at[slot].at[pl.ds(g * 16, 16)],
                sems.at[lax.rem(g, NSEM)],
            )
            return carry

        lax.fori_loop(0, VG, one, 0)

    def drain(slot):
        rows = (VG // NSEM) * 16
        for k in range(NSEM):
            pltpu.make_async_copy(
                table_hbm.at[idx_v.at[0].at[pl.ds(0, rows)]],
                buf_v.at[slot].at[pl.ds(0, rows)],
                sems.at[k],
            ).wait()

    start_gathers(0, 0)

    def chunk_body(ch, carry):
        slot = lax.rem(ch, 2)
        nslot = lax.rem(ch + 1, 2)

        @pl.when(ch + 1 < NCHUNK)
        def _():
            start_gathers(ch + 1, nslot)

        drain(slot)

        def red_body(j, carry2):
            a0 = buf_v[slot, j, pl.ds(0, 16)]
            a1 = buf_v[slot, j, pl.ds(16, 16)]
            for f in range(1, NUM_FIELDS):
                row = f * CB + j
                a0 = a0 + buf_v[slot, row, pl.ds(0, 16)]
                a1 = a1 + buf_v[slot, row, pl.ds(16, 16)]
            outb_v[j, pl.ds(0, 16)] = a0
            outb_v[j, pl.ds(16, 16)] = a1
            return carry2

        lax.fori_loop(0, CB, red_body, 0)

        pltpu.sync_copy(outb_v, out_hbm.at[pl.ds(base + ch * CB, CB)])
        return carry

    lax.fori_loop(0, NCHUNK, chunk_body, 0)


_emb_call = functools.partial(
    pl.kernel,
    mesh=plsc.VectorSubcoreMesh(
        core_axis_name="c", subcore_axis_name="s", num_cores=NC, num_subcores=NS
    ),
    out_type=jax.ShapeDtypeStruct((BATCH, EMB_DIM), jnp.float32),
    scratch_types=[
        pltpu.VMEM((NCHUNK, ROWS_PER_CHUNK), jnp.int32),
        pltpu.VMEM((2, ROWS_PER_CHUNK, EMB_DIM), jnp.float32),
        pltpu.VMEM((CB, EMB_DIM), jnp.float32),
        pltpu.SemaphoreType.DMA((NSEM,)),
    ],
    compiler_params=pltpu.CompilerParams(use_tc_tiling_on_sc=False),
)(_sc_body)


@jax.jit
def kernel(g, x, tables):
    x = x.astype(jnp.int32)
    offs = (jnp.arange(NUM_FIELDS, dtype=jnp.int32) * VOCAB)[None, :]
    flat = x + offs
    flat = flat.reshape(NW, NCHUNK, CB, NUM_FIELDS).transpose(0, 1, 3, 2)
    idx = flat.reshape(NW, NCHUNK, ROWS_PER_CHUNK)
    table = tables.reshape(NUM_FIELDS * VOCAB, EMB_DIM)
    return _emb_call(idx, table)


# speedup vs baseline: 1.3310x; 1.2342x over previous
import functools

import jax
import jax.numpy as jnp
from jax import lax
from jax.experimental import pallas as pl
from jax.experimental.pallas import tpu as pltpu
from jax.experimental.pallas import tpu_sc as plsc

NUM_FIELDS = 26
VOCAB = 100000
EMB_DIM = 32
BATCH = 16384

NC = 2
NS = 16
NW = NC * NS
BPW = BATCH // NW
NPLANES = NUM_FIELDS * EMB_DIM
RING = 8


def _sc_body(idx_hbm, t2_hbm, out_hbm, idx_v, pbuf_v, acc_v, sems):
    c = lax.axis_index("c")
    s = lax.axis_index("s")
    wid = s * NC + c
    base = wid * BPW

    pltpu.sync_copy(idx_hbm.at[wid], idx_v)

    def fire(p):
        f = lax.div(p, EMB_DIM)
        slot = lax.rem(p, RING)
        pltpu.async_copy(
            t2_hbm.at[p].at[idx_v.at[f]], pbuf_v.at[slot], sems.at[slot]
        )

    def drain_one(slot):
        pltpu.make_async_copy(
            t2_hbm.at[0].at[idx_v.at[0]], pbuf_v.at[0], sems.at[slot]
        ).wait()

    def prime(p, carry):
        fire(p)
        return carry

    lax.fori_loop(0, RING, prime, 0)

    def plane_body(p, carry):
        d = lax.rem(p, EMB_DIM)
        slot = lax.rem(p, RING)
        fld = lax.div(p, EMB_DIM)
        drain_one(slot)

        @pl.when(fld == 0)
        def _():
            for k in range(BPW // 16):
                acc_v[d, pl.ds(k * 16, 16)] = pbuf_v[slot, pl.ds(k * 16, 16)]

        @pl.when(fld > 0)
        def _():
            for k in range(BPW // 16):
                plsc.addupdate(
                    acc_v.at[d].at[pl.ds(k * 16, 16)],
                    pbuf_v[slot, pl.ds(k * 16, 16)],
                )

        @pl.when(p + RING < NPLANES)
        def _():
            fire(p + RING)

        return carry

    lax.fori_loop(0, NPLANES, plane_body, 0)

    for dd in range(EMB_DIM):
        pltpu.sync_copy(acc_v.at[dd], out_hbm.at[dd].at[pl.ds(base, BPW)])


_emb_call = functools.partial(
    pl.kernel,
    mesh=plsc.VectorSubcoreMesh(
        core_axis_name="c", subcore_axis_name="s", num_cores=NC, num_subcores=NS
    ),
    out_type=jax.ShapeDtypeStruct((EMB_DIM, BATCH), jnp.float32),
    scratch_types=[
        pltpu.VMEM((NUM_FIELDS, BPW), jnp.int32),
        pltpu.VMEM((RING, BPW), jnp.float32),
        pltpu.VMEM((EMB_DIM, BPW), jnp.float32),
        pltpu.SemaphoreType.DMA((RING,)),
    ],
    compiler_params=pltpu.CompilerParams(use_tc_tiling_on_sc=False),
)(_sc_body)


@jax.jit
def kernel(g, x, tables):
    x = x.astype(jnp.int32)
    t2 = jnp.transpose(tables, (0, 2, 1)).reshape(NPLANES, VOCAB)
    idx = x.reshape(NW, BPW, NUM_FIELDS).transpose(0, 2, 1)
    out = _emb_call(idx, t2)
    return out.T
